# baseline (speedup 1.0000x reference)
import functools

import jax
import jax.numpy as jnp
from jax import lax
from jax.experimental import pallas as pl
from jax.experimental.pallas import tpu as pltpu
from jax.experimental.pallas import tpu_sc as plsc

_OUT_DIM = 5
_ETA_MIN = 0.8
_ETA_RANGE = 0.99 - 0.8

_B, _T = 16384, 200
_N = _B * _T
_LANES = 128
_ROWS = _N // _LANES
_NC, _NS = 2, 16
_NW = _NC * _NS
_RPW = _ROWS // _NW
_CH = 16
_NCH = _RPW // _CH
_NROW = _CH * _LANES
_V = 100000
_DP = 8


def _sigmoid_body(x_ref, o_ref):
    x = x_ref[...]
    o_ref[...] = _ETA_MIN + _ETA_RANGE / (1.0 + jnp.exp(-x))


_transform = pl.pallas_call(
    _sigmoid_body,
    out_shape=jax.ShapeDtypeStruct((_V * _DP // _LANES, _LANES), jnp.float32),
)

_sc_mesh = plsc.VectorSubcoreMesh(core_axis_name="c", subcore_axis_name="s")


@functools.partial(
    pl.kernel,
    mesh=_sc_mesh,
    out_type=jax.ShapeDtypeStruct((_N, _OUT_DIM), jnp.float32),
    scratch_types=[
        pltpu.VMEM((_NROW,), jnp.int32),
        pltpu.VMEM((_NROW, _DP), jnp.float32),
        pltpu.VMEM_SHARED((_V, _DP), jnp.float32),
        pltpu.SemaphoreType.DMA,
    ],
    compiler_params=pltpu.CompilerParams(use_tc_tiling_on_sc=False),
)
def _gather_kernel(table_hbm, idx_hbm, out_hbm, idx_v, rows_v, table_sp, sem):
    sid = lax.axis_index("s")
    wid = sid * _NC + lax.axis_index("c")

    @pl.when(sid == 0)
    def _stage():
        pltpu.sync_copy(table_hbm, table_sp)

    plsc.subcore_barrier()

    def chunk(k, carry):
        e0 = (wid * _NCH + k) * _NROW
        pltpu.sync_copy(idx_hbm.at[pl.ds(e0, _NROW)], idx_v)
        pltpu.async_copy(table_sp.at[idx_v], rows_v, sem).wait()
        pltpu.sync_copy(rows_v.at[:, pl.ds(0, _OUT_DIM)],
                        out_hbm.at[pl.ds(e0, _NROW)])
        return carry

    lax.fori_loop(0, _NCH, chunk, 0)


def kernel(ops_t, cond_ids, eta_table):
    del ops_t
    padded = jnp.pad(eta_table, ((0, 0), (0, _DP - _OUT_DIM)))
    table = _transform(padded.reshape(-1, _LANES)).reshape(_V, _DP)
    idx = cond_ids.reshape(_N)
    out = _gather_kernel(table, idx)
    return out.reshape(_B, _T, _OUT_DIM)


# speedup vs baseline: 3.8589x; 3.8589x over previous
import functools

import jax
import jax.numpy as jnp
from jax import lax
from jax.experimental import pallas as pl
from jax.experimental.pallas import tpu as pltpu
from jax.experimental.pallas import tpu_sc as plsc

_OUT_DIM = 5
_ETA_MIN = 0.8
_ETA_RANGE = 0.99 - 0.8

_B, _T = 16384, 200
_N = _B * _T
_LANES = 128
_ROWS = _N // _LANES
_NC, _NS = 2, 16
_NW = _NC * _NS
_RPW = _ROWS // _NW
_CH = 16
_NCH = _RPW // _CH
_NROW = _CH * _LANES
_V = 100000
_DP = 8


def _sigmoid_body(x_ref, o_ref):
    x = x_ref[...]
    o_ref[...] = _ETA_MIN + _ETA_RANGE / (1.0 + jnp.exp(-x))


_transform = pl.pallas_call(
    _sigmoid_body,
    out_shape=jax.ShapeDtypeStruct((_V * _DP // _LANES, _LANES), jnp.float32),
)

_sc_mesh = plsc.VectorSubcoreMesh(core_axis_name="c", subcore_axis_name="s")


@functools.partial(
    pl.kernel,
    mesh=_sc_mesh,
    out_type=jax.ShapeDtypeStruct((_N, _DP), jnp.float32),
    scratch_types=[
        pltpu.VMEM((_NROW,), jnp.int32),
        pltpu.VMEM((_NROW, _DP), jnp.float32),
        pltpu.VMEM_SHARED((_V, _DP), jnp.float32),
        pltpu.SemaphoreType.DMA,
    ],
    compiler_params=pltpu.CompilerParams(use_tc_tiling_on_sc=False),
)
def _gather_kernel(table_hbm, idx_hbm, out_hbm, idx_v, rows_v, table_sp, sem):
    sid = lax.axis_index("s")
    wid = sid * _NC + lax.axis_index("c")

    @pl.when(sid == 0)
    def _stage():
        pltpu.sync_copy(table_hbm, table_sp)

    plsc.subcore_barrier()

    def chunk(k, carry):
        e0 = (wid * _NCH + k) * _NROW
        pltpu.sync_copy(idx_hbm.at[pl.ds(e0, _NROW)], idx_v)
        pltpu.async_copy(table_sp.at[idx_v], rows_v, sem).wait()
        pltpu.sync_copy(rows_v, out_hbm.at[pl.ds(e0, _NROW)])
        return carry

    lax.fori_loop(0, _NCH, chunk, 0)


def kernel(ops_t, cond_ids, eta_table):
    del ops_t
    padded = jnp.pad(eta_table, ((0, 0), (0, _DP - _OUT_DIM)))
    table = _transform(padded.reshape(-1, _LANES)).reshape(_V, _DP)
    idx = cond_ids.reshape(_N)
    out = _gather_kernel(table, idx)[:, :_OUT_DIM]
    return out.reshape(_B, _T, _OUT_DIM)


# speedup vs baseline: 28.8567x; 7.4780x over previous
import functools

import jax
import jax.numpy as jnp
from jax import lax
from jax.experimental import pallas as pl
from jax.experimental.pallas import tpu as pltpu
from jax.experimental.pallas import tpu_sc as plsc

_D = 5
_ETA_MIN = 0.8
_ETA_RANGE = 0.99 - 0.8

_B, _T = 16384, 200
_N = _B * _T
_V = 100000
_FLAT = _V * _D
_FPAD = 3907 * 128

_NC, _NS = 2, 16
_NW = _NC * _NS
_TT = _T // 8
_BB = _B // 128
_CHIDS = 8 * 128
_NCH = _TT * _BB
_CPW = _NCH // _NW


def _sigmoid_body(x_ref, o_ref):
    x = x_ref[...]
    o_ref[...] = _ETA_MIN + _ETA_RANGE / (1.0 + jnp.exp(-x))


_transform = pl.pallas_call(
    _sigmoid_body,
    out_shape=jax.ShapeDtypeStruct((_FPAD // 128, 128), jnp.float32),
)

_sc_mesh = plsc.VectorSubcoreMesh(core_axis_name="c", subcore_axis_name="s")


@functools.partial(
    pl.kernel,
    mesh=_sc_mesh,
    out_type=jax.ShapeDtypeStruct((_NCH * _D * _CHIDS,), jnp.float32),
    scratch_types=[
        pltpu.VMEM((_CHIDS,), jnp.int32),
        pltpu.VMEM((_D * _CHIDS,), jnp.float32),
        pltpu.VMEM_SHARED((_D, _V), jnp.float32),
        pltpu.SemaphoreType.DMA,
    ],
    compiler_params=pltpu.CompilerParams(use_tc_tiling_on_sc=False),
)
def _gather_kernel(table_hbm, idx_hbm, out_hbm, idx_v, tile_v, table_sp, sem):
    sid = lax.axis_index("s")
    wid = sid * _NC + lax.axis_index("c")

    @pl.when(sid == 0)
    def _stage():
        pltpu.sync_copy(table_hbm, table_sp)

    plsc.subcore_barrier()

    def chunk(k, carry):
        ct = wid * _CPW + k
        tt = ct // _BB
        bb = ct - tt * _BB
        pltpu.sync_copy(idx_hbm.at[pl.ds(ct * _CHIDS, _CHIDS)], idx_v)
        gathers = [
            pltpu.async_copy(table_sp.at[c].at[idx_v],
                             tile_v.at[pl.ds(c * _CHIDS, _CHIDS)], sem)
            for c in range(_D)
        ]
        for g in gathers:
            g.wait()
        writes = [
            pltpu.async_copy(
                tile_v.at[pl.ds(c * _CHIDS, _CHIDS)],
                out_hbm.at[pl.ds(((c * _TT + tt) * _BB + bb) * _CHIDS,
                                 _CHIDS)], sem)
            for c in range(_D)
        ]
        for w in writes:
            w.wait()
        return carry

    lax.fori_loop(0, _CPW, chunk, 0)


def kernel(ops_t, cond_ids, eta_table):
    del ops_t
    flat_cm = jnp.pad(eta_table.T.reshape(-1), (0, _FPAD - _FLAT))
    table = _transform(flat_cm.reshape(-1, 128)).reshape(-1)[:_FLAT]
    table = table.reshape(_D, _V)
    idx = (cond_ids.T.reshape(_TT, 8, _BB, 128)
           .transpose(0, 2, 1, 3).reshape(_N))
    out = _gather_kernel(table, idx)
    out = out.reshape(_D, _TT, _BB, 8, 128).transpose(2, 4, 1, 3, 0)
    return out.reshape(_B, _T, _D)


# speedup vs baseline: 35.9544x; 1.2460x over previous
import functools

import jax
import jax.numpy as jnp
from jax import lax
from jax.experimental import pallas as pl
from jax.experimental.pallas import tpu as pltpu
from jax.experimental.pallas import tpu_sc as plsc

_D = 5
_ETA_MIN = 0.8
_ETA_RANGE = 0.99 - 0.8

_B, _T = 16384, 200
_N = _B * _T
_V = 100000
_FLAT = _V * _D
_FPAD = 3907 * 128

_NC, _NS = 2, 16
_NW = _NC * _NS
_TT = _T // 8
_BB = _B // 128
_QT = 4
_CHIDS = 8 * 128 * _QT
_NCH = _TT * _BB // _QT
_CPW = _NCH // _NW


def _sigmoid_body(x_ref, o_ref):
    x = x_ref[...]
    o_ref[...] = _ETA_MIN + _ETA_RANGE / (1.0 + jnp.exp(-x))


_transform = pl.pallas_call(
    _sigmoid_body,
    out_shape=jax.ShapeDtypeStruct((_FPAD // 128, 128), jnp.float32),
)

_sc_mesh = plsc.VectorSubcoreMesh(core_axis_name="c", subcore_axis_name="s")


@functools.partial(
    pl.kernel,
    mesh=_sc_mesh,
    out_type=jax.ShapeDtypeStruct((_D * _N,), jnp.float32),
    scratch_types=[
        pltpu.VMEM((_CHIDS,), jnp.int32),
        pltpu.VMEM((_D * _CHIDS,), jnp.float32),
        pltpu.VMEM_SHARED((_D, _V), jnp.float32),
        pltpu.SemaphoreType.DMA,
    ],
    compiler_params=pltpu.CompilerParams(use_tc_tiling_on_sc=False),
)
def _gather_kernel(table_hbm, idx_hbm, out_hbm, idx_v, tile_v, table_sp, sem):
    sid = lax.axis_index("s")
    wid = sid * _NC + lax.axis_index("c")

    @pl.when(sid == 0)
    def _stage():
        pltpu.sync_copy(table_hbm, table_sp)

    plsc.subcore_barrier()

    def chunk(k, carry):
        ct = wid * _CPW + k
        nq = _BB // _QT
        tt = ct // nq
        bb = (ct - tt * nq) * _QT
        pltpu.sync_copy(idx_hbm.at[pl.ds(ct * _CHIDS, _CHIDS)], idx_v)
        gathers = [
            pltpu.async_copy(table_sp.at[c].at[idx_v],
                             tile_v.at[pl.ds(c * _CHIDS, _CHIDS)], sem)
            for c in range(_D)
        ]
        for g in gathers:
            g.wait()
        writes = [
            pltpu.async_copy(
                tile_v.at[pl.ds(c * _CHIDS, _CHIDS)],
                out_hbm.at[pl.ds(((c * _TT + tt) * _BB + bb) * 1024,
                                 _CHIDS)], sem)
            for c in range(_D)
        ]
        for w in writes:
            w.wait()
        return carry

    lax.fori_loop(0, _CPW, chunk, 0)


def kernel(ops_t, cond_ids, eta_table):
    del ops_t
    flat_cm = jnp.pad(eta_table.T.reshape(-1), (0, _FPAD - _FLAT))
    table = _transform(flat_cm.reshape(-1, 128)).reshape(-1)[:_FLAT]
    table = table.reshape(_D, _V)
    idx = (cond_ids.T.reshape(_TT, 8, _BB, 128)
           .transpose(0, 2, 1, 3).reshape(_N))
    out = _gather_kernel(table, idx)
    out = out.reshape(_D, _TT, _BB, 8, 128).transpose(2, 4, 1, 3, 0)
    return out.reshape(_B, _T, _D)
